# Initial kernel scaffold; baseline (speedup 1.0000x reference)
import jax
import jax.numpy as jnp
from jax.experimental import pallas as pl


def kernel(costh, label):
    raise NotImplementedError("write your pallas kernel here")


# speedup vs baseline: 2.2252x; 2.2252x over previous
import jax
import jax.numpy as jnp
from jax.experimental import pallas as pl
from jax.experimental.pallas import tpu as pltpu

_B = 1024
_C = 100000
_W = 2048
_NBLK = (_C + _W - 1) // _W


def _loss_kernel(costh_ref, label_ref, out_ref, se_acc, pk_acc):
    jb = pl.program_id(0)

    @pl.when(jb == 0)
    def _init():
        se_acc[...] = jnp.zeros_like(se_acc)
        pk_acc[...] = jnp.zeros_like(pk_acc)

    c = costh_ref[...]
    cols = jb * _W + jax.lax.broadcasted_iota(jnp.int32, (_B, _W), 1)
    lab = label_ref[...]
    is_lab = cols == lab
    a = 8.5 * c - jnp.where(is_lab, 2.25, 0.0)
    e = jnp.where(cols < _C, jnp.exp(a), 0.0)
    se_acc[...] += jnp.sum(e, axis=1, keepdims=True)
    pk_acc[...] += jnp.sum(jnp.where(is_lab, a, 0.0), axis=1, keepdims=True)

    @pl.when(jb == _NBLK - 1)
    def _fin():
        loss_i = jnp.log(se_acc[...]) - pk_acc[...]
        out_ref[...] = jnp.mean(loss_i, keepdims=True)


def kernel(costh, label):
    label2d = label.astype(jnp.int32).reshape(_B, 1)
    out = pl.pallas_call(
        _loss_kernel,
        grid=(_NBLK,),
        in_specs=[
            pl.BlockSpec((_B, _W), lambda j: (0, j)),
            pl.BlockSpec((_B, 1), lambda j: (0, 0)),
        ],
        out_specs=pl.BlockSpec((1, 1), lambda j: (0, 0)),
        out_shape=jax.ShapeDtypeStruct((1, 1), jnp.float32),
        scratch_shapes=[
            pltpu.VMEM((_B, 1), jnp.float32),
            pltpu.VMEM((_B, 1), jnp.float32),
        ],
    )(costh, label2d)
    return out[0, 0]
